# baseline (speedup 1.0000x reference)
import jax
import jax.numpy as jnp
from jax import lax
from jax.experimental import pallas as pl
from jax.experimental.pallas import tpu as pltpu


def _assemble_kernel(w_hbm, out_hbm, scr_w, scr_out, sem_i, sem_o):
    ci = pltpu.make_async_copy(w_hbm, scr_w, sem_i)
    ci.start()
    row = lax.broadcasted_iota(jnp.int32, (6, 4), 0)
    col = lax.broadcasted_iota(jnp.int32, (6, 4), 1)
    fixed = jnp.where(row == 5, 0.25,
                      jnp.where(row - 1 == col, 1.0, 0.0)).astype(jnp.float32)
    ci.wait()
    scr_out[...] = jnp.concatenate([fixed, scr_w[...]], axis=0)
    co = pltpu.make_async_copy(scr_out, out_hbm, sem_o)
    co.start()
    co.wait()


def kernel(RNA_embedding_weight):
    return pl.pallas_call(
        _assemble_kernel,
        in_specs=[pl.BlockSpec(memory_space=pl.ANY)],
        out_specs=pl.BlockSpec(memory_space=pl.ANY),
        out_shape=jax.ShapeDtypeStruct((19, 4), jnp.float32),
        scratch_shapes=[
            pltpu.VMEM((13, 4), jnp.float32),
            pltpu.VMEM((19, 4), jnp.float32),
            pltpu.SemaphoreType.DMA,
            pltpu.SemaphoreType.DMA,
        ],
    )(RNA_embedding_weight)
